# baseline (speedup 1.0000x reference)
import jax
import jax.numpy as jnp
from jax import lax
from jax.experimental import pallas as pl
from jax.experimental.pallas import tpu as pltpu

_B, _L, _C = 4096, 200, 21
_BB = 512

_SPECIAL = {2: 22, 11: 22, 3: 23, 13: 23, 7: 24, 9: 24}


def _body(seq_ref, out_ref):
    s = seq_ref[...]
    half = {
        22: jnp.where(s == 22, 0.5, 0.0),
        23: jnp.where(s == 23, 0.5, 0.0),
        24: jnp.where(s == 24, 0.5, 0.0),
    }
    for c in range(_C):
        v = jnp.where(s == c + 1, 1.0, 0.0)
        if c in _SPECIAL:
            v = v + half[_SPECIAL[c]]
        out_ref[c, :, :] = v


def kernel(sequence):
    seq_t = sequence.T
    out_t = pl.pallas_call(
        _body,
        grid=(_B // _BB,),
        in_specs=[pl.BlockSpec((_L, _BB), lambda i: (0, i))],
        out_specs=pl.BlockSpec((_C, _L, _BB), lambda i: (0, 0, i)),
        out_shape=jax.ShapeDtypeStruct((_C, _L, _B), jnp.float32),
        compiler_params=pltpu.CompilerParams(
            dimension_semantics=("parallel",)
        ),
    )(seq_t)
    return out_t.transpose(2, 1, 0)
